# Initial kernel scaffold; baseline (speedup 1.0000x reference)
import jax
import jax.numpy as jnp
from jax.experimental import pallas as pl


def kernel(x, edge_index, edge_attr, waypoints, conv_params, bn_gamma, bn_beta, fc1_W, fc1_b, fc2_W, fc2_b):
    raise NotImplementedError("write your pallas kernel here")


# speedup vs baseline: 1.0839x; 1.0839x over previous
import functools

import jax
import jax.numpy as jnp
from jax import lax
from jax.experimental import pallas as pl
from jax.experimental.pallas import tpu as pltpu

N_NODES = 10000
NB_WAY = 64


def _sigmoid(x):
    return 1.0 / (1.0 + jnp.exp(-x))


def _final_body(h_ref, wp_ref, gamma_ref, beta_ref, fc1a_ref, fc1b_ref,
                fc2w_ref, fc2b_ref, out_ref):
    h = h_ref[...]
    n = h.shape[0]
    cf = h.shape[1]
    s = jnp.sum(h, axis=0)
    ss = jnp.sum(h * h, axis=0)
    mean = s / n
    var = ss / n - mean * mean
    inv = (gamma_ref[0, :] / jnp.sqrt(var + 1e-5))

    wp = wp_ref[...]
    cols = lax.broadcasted_iota(jnp.int32, (NB_WAY, n), 1)
    onehot = jnp.where(cols == wp, 1.0, 0.0).astype(jnp.float32)
    sel = jnp.dot(onehot, h, preferred_element_type=jnp.float32)
    sel = (sel - mean) * inv + beta_ref[0, :]

    fc1 = fc1a_ref[...]
    A = jnp.dot(sel, fc1[:cf], preferred_element_type=jnp.float32)
    A = A + fc1b_ref[0, :]
    B = jnp.dot(sel, fc1[cf:], preferred_element_type=jnp.float32)

    fc2w = fc2w_ref[...]
    fc2b = fc2b_ref[0, :]
    CH = 8
    rows = []
    for c in range(NB_WAY // CH):
        Ac = A[c * CH:(c + 1) * CH]
        hid = Ac[:, None, :] + B[None, :, :]
        hid = _sigmoid(hid).reshape(CH * NB_WAY, A.shape[1])
        p = jnp.dot(hid, fc2w, preferred_element_type=jnp.float32) + fc2b
        rows.append(jnp.max(p, axis=1).reshape(CH, NB_WAY))
    t = _sigmoid(jnp.concatenate(rows, axis=0))
    eye = (lax.broadcasted_iota(jnp.int32, (NB_WAY, NB_WAY), 0)
           == lax.broadcasted_iota(jnp.int32, (NB_WAY, NB_WAY), 1))
    t = jnp.where(eye, 0.0, t)
    out_ref[0, :, :] = t.T


def _final_tail(h, waypoints, bn_gamma, bn_beta, fc1_W, fc1_b, fc2_W, fc2_b):
    cf = h.shape[1]
    wp2 = waypoints.astype(jnp.int32).reshape(NB_WAY, 1)
    fc2w_p = jnp.pad(fc2_W, ((0, 0), (0, 28)))
    fc2b_p = jnp.pad(fc2_b, (0, 28), constant_values=-jnp.inf)
    return pl.pallas_call(
        _final_body,
        out_shape=jax.ShapeDtypeStruct((1, NB_WAY, NB_WAY), jnp.float32),
    )(h, wp2, bn_gamma.reshape(1, cf), bn_beta.reshape(1, cf),
      fc1_W, fc1_b.reshape(1, -1), fc2w_p, fc2b_p.reshape(1, -1))


def kernel(x, edge_index, edge_attr, waypoints, conv_params, bn_gamma,
           bn_beta, fc1_W, fc1_b, fc2_W, fc2_b):
    src = edge_index[0]
    dst = edge_index[1]
    h = x
    for W_rel, b_rel, W_root in conv_params:
        prev, size = W_rel.shape
        if prev > size:
            m = h @ W_rel
            agg = jax.ops.segment_sum(m[src] * edge_attr[:, None], dst,
                                      num_segments=N_NODES)
            h = _sigmoid(agg + b_rel + h @ W_root)
        else:
            agg = jax.ops.segment_sum(h[src] * edge_attr[:, None], dst,
                                      num_segments=N_NODES)
            h = _sigmoid(agg @ W_rel + b_rel + h @ W_root)
    return _final_tail(h, waypoints, bn_gamma, bn_beta, fc1_W, fc1_b,
                       fc2_W, fc2_b)


# speedup vs baseline: 7.6599x; 7.0668x over previous
import functools

import jax
import jax.numpy as jnp
from jax import lax
from jax.experimental import pallas as pl
from jax.experimental.pallas import tpu as pltpu
from jax.experimental.pallas import tpu_sc as plsc

N_NODES = 10000
NB_WAY = 64
NSUB = 16
GRP = 160
GB = 128
E_PAD = NSUB * GRP * GB
N_ACC = 10240
ROWS_PER_SUB = N_ACC // NSUB
CHG = 32
NCH = GRP // CHG


def _sigmoid(x):
    return 1.0 / (1.0 + jnp.exp(-x))


def _pad16(n):
    return -(-n // 16) * 16


def _half_w(wpad):
    return 16 if wpad <= 32 else 80


@functools.lru_cache(maxsize=None)
def _make_edge_agg(wh):
    nj = wh // 16
    mesh = plsc.VectorSubcoreMesh(core_axis_name="c", subcore_axis_name="s")

    @functools.partial(
        pl.kernel, mesh=mesh,
        compiler_params=pltpu.CompilerParams(use_tc_tiling_on_sc=False),
        out_type=[jax.ShapeDtypeStruct((N_ACC, wh), jnp.float32),
                  jax.ShapeDtypeStruct((N_ACC, wh), jnp.float32)],
        scratch_types=[
            pltpu.VMEM((CHG, GB), jnp.int32),
            pltpu.VMEM((CHG, GB), jnp.int32),
            pltpu.VMEM((CHG, GB), jnp.float32),
            pltpu.VMEM((GB, wh), jnp.float32),
            pltpu.VMEM((GB, wh), jnp.float32),
            pltpu.VMEM_SHARED((N_ACC, wh), jnp.float32),
            pltpu.SemaphoreType.DMA,
            pltpu.SemaphoreType.DMA,
        ],
    )
    def edge_agg(mlo_hbm, mhi_hbm, src_hbm, dst_hbm, attr_hbm,
                 out_lo, out_hi, src_v, dst_v, attr_v, rows0, rows1,
                 agg_sh, sem0, sem1):
        cid = lax.axis_index("c")
        sid = lax.axis_index("s")

        zero16 = jnp.zeros((16,), jnp.float32)

        def zrow(i, carry):
            for j in range(nj):
                rows0[i, pl.ds(j * 16, 16)] = zero16
            return carry

        lax.fori_loop(0, GB, zrow, 0)
        base = sid * ROWS_PER_SUB
        for q in range(ROWS_PER_SUB // GB):
            pltpu.sync_copy(rows0, agg_sh.at[pl.ds(base + q * GB, GB)])
        plsc.subcore_barrier()

        def scale(rows, g):
            def body(bb, carry):
                av = attr_v[g, pl.ds(bb * 16, 16)]
                for e in range(16):
                    a16 = jnp.full((16,), av[e], jnp.float32)
                    b = bb * 16 + e
                    for j in range(nj):
                        rows[b, pl.ds(j * 16, 16)] = (
                            rows[b, pl.ds(j * 16, 16)] * a16)
                return carry
            lax.fori_loop(0, GB // 16, body, 0)

        def main_loop(m_hbm):
            def chunk(c, carry):
                pltpu.sync_copy(src_hbm.at[sid, pl.ds(c * CHG, CHG)], src_v)
                pltpu.sync_copy(dst_hbm.at[sid, pl.ds(c * CHG, CHG)], dst_v)
                pltpu.sync_copy(attr_hbm.at[sid, pl.ds(c * CHG, CHG)], attr_v)
                pltpu.async_copy(m_hbm.at[src_v.at[0]], rows0, sem0)

                def step(p, carry):
                    g0 = 2 * p
                    g1 = 2 * p + 1
                    pltpu.make_async_copy(
                        m_hbm.at[src_v.at[g0]], rows0, sem0).wait()
                    pltpu.async_copy(m_hbm.at[src_v.at[g1]], rows1, sem1)
                    scale(rows0, g0)
                    pltpu.sync_copy(rows0, agg_sh.at[dst_v.at[g0]], add=True)
                    g2 = jnp.where(g1 + 1 < CHG, g1 + 1, 0)
                    pltpu.make_async_copy(
                        m_hbm.at[src_v.at[g1]], rows1, sem1).wait()
                    pltpu.async_copy(m_hbm.at[src_v.at[g2]], rows0, sem0)
                    scale(rows1, g1)
                    pltpu.sync_copy(rows1, agg_sh.at[dst_v.at[g1]], add=True)
                    return carry

                lax.fori_loop(0, CHG // 2, step, 0)
                pltpu.make_async_copy(
                    m_hbm.at[src_v.at[0]], rows0, sem0).wait()
                return carry

            lax.fori_loop(0, NCH, chunk, 0)

        @pl.when(cid == 0)
        def _():
            main_loop(mlo_hbm)

        @pl.when(cid == 1)
        def _():
            main_loop(mhi_hbm)

        plsc.subcore_barrier()

        @pl.when(cid == 0)
        def _():
            pltpu.sync_copy(agg_sh.at[pl.ds(base, ROWS_PER_SUB)],
                            out_lo.at[pl.ds(base, ROWS_PER_SUB)])

        @pl.when(cid == 1)
        def _():
            pltpu.sync_copy(agg_sh.at[pl.ds(base, ROWS_PER_SUB)],
                            out_hi.at[pl.ds(base, ROWS_PER_SUB)])

    return edge_agg


def _proj0_body(x_ref, wrel_ref, wroot_ref, m_ref, r_ref):
    x = x_ref[...]
    m_ref[...] = jnp.dot(x, wrel_ref[...], preferred_element_type=jnp.float32)
    r_ref[...] = jnp.dot(x, wroot_ref[...], preferred_element_type=jnp.float32)


def _l0post_body(agg_ref, r_ref, b_ref, out_ref):
    out_ref[...] = _sigmoid(agg_ref[:N_NODES] + r_ref[...] + b_ref[0, :])


def _dense_body(wh_out, alo_ref, ahi_ref, hlo_ref, hhi_ref, wrl_ref, wrh_ref,
                wtl_ref, wth_ref, b_ref, olo_ref, ohi_ref):
    z = (jnp.dot(alo_ref[:N_NODES], wrl_ref[...],
                 preferred_element_type=jnp.float32)
         + jnp.dot(ahi_ref[:N_NODES], wrh_ref[...],
                   preferred_element_type=jnp.float32)
         + jnp.dot(hlo_ref[...], wtl_ref[...],
                   preferred_element_type=jnp.float32)
         + jnp.dot(hhi_ref[...], wth_ref[...],
                   preferred_element_type=jnp.float32)
         + b_ref[0, :])
    h = _sigmoid(z)
    wpad = z.shape[1]
    n = h.shape[0]
    lo_w = min(wh_out, wpad)
    lo = h[:, :lo_w]
    if lo_w < wh_out:
        lo = jnp.concatenate(
            [lo, jnp.zeros((n, wh_out - lo_w), jnp.float32)], axis=1)
    olo_ref[...] = lo
    if wpad > wh_out:
        hi = h[:, wh_out:]
        if wpad - wh_out < wh_out:
            hi = jnp.concatenate(
                [hi, jnp.zeros((n, 2 * wh_out - wpad), jnp.float32)], axis=1)
    else:
        hi = jnp.zeros((n, wh_out), jnp.float32)
    ohi_ref[...] = hi


def _final_body(hlo_ref, hhi_ref, wp_ref, gamma_ref, beta_ref, fc1a_ref,
                fc1b_ref, fc2w_ref, fc2b_ref, out_ref):
    h = jnp.concatenate([hlo_ref[...], hhi_ref[...]], axis=1)
    n = h.shape[0]
    cf = h.shape[1]
    mean = jnp.sum(h, axis=0) / n
    var = jnp.sum(h * h, axis=0) / n - mean * mean
    inv = gamma_ref[0, :] / jnp.sqrt(var + 1e-5)

    wp = wp_ref[...]
    cols = lax.broadcasted_iota(jnp.int32, (NB_WAY, n), 1)
    onehot = jnp.where(cols == wp, 1.0, 0.0).astype(jnp.float32)
    sel = jnp.dot(onehot, h, preferred_element_type=jnp.float32)
    sel = (sel - mean) * inv + beta_ref[0, :]

    fc1 = fc1a_ref[...]
    A = jnp.dot(sel, fc1[:cf], preferred_element_type=jnp.float32)
    A = A + fc1b_ref[0, :]
    B = jnp.dot(sel, fc1[cf:], preferred_element_type=jnp.float32)

    fc2w = fc2w_ref[...]
    fc2b = fc2b_ref[0, :]
    CH = 8
    rows = []
    for c in range(NB_WAY // CH):
        Ac = A[c * CH:(c + 1) * CH]
        hid = Ac[:, None, :] + B[None, :, :]
        hid = _sigmoid(hid).reshape(CH * NB_WAY, A.shape[1])
        p = jnp.dot(hid, fc2w, preferred_element_type=jnp.float32) + fc2b
        rows.append(jnp.max(p, axis=1).reshape(CH, NB_WAY))
    t = _sigmoid(jnp.concatenate(rows, axis=0))
    eye = (lax.broadcasted_iota(jnp.int32, (NB_WAY, NB_WAY), 0)
           == lax.broadcasted_iota(jnp.int32, (NB_WAY, NB_WAY), 1))
    t = jnp.where(eye, 0.0, t)
    out_ref[0, :, :] = t.T


def _split_cols(M, wh):
    rows = M.shape[0]
    lo = M[:min(wh, rows)]
    if lo.shape[0] < wh:
        lo = jnp.pad(lo, ((0, wh - lo.shape[0]), (0, 0)))
    if rows > wh:
        hi = jnp.pad(M[wh:], ((0, 2 * wh - rows), (0, 0)))
    else:
        hi = jnp.zeros((wh,) + M.shape[1:], M.dtype)
    return lo, hi


def kernel(x, edge_index, edge_attr, waypoints, conv_params, bn_gamma,
           bn_beta, fc1_W, fc1_b, fc2_W, fc2_b):
    src = edge_index[0].astype(jnp.int32)
    dst = edge_index[1].astype(jnp.int32)
    npad = E_PAD - src.shape[0]
    pad_idx = (jnp.arange(npad, dtype=jnp.int32) * 97) % N_NODES
    srcp = jnp.concatenate([src, pad_idx]).reshape(NSUB, GRP, GB)
    dstp = jnp.concatenate([dst, pad_idx]).reshape(NSUB, GRP, GB)
    attrp = jnp.concatenate(
        [edge_attr, jnp.zeros((npad,), jnp.float32)]).reshape(NSUB, GRP, GB)

    W0, b0, R0 = conv_params[0]
    s0 = W0.shape[1]
    wp0 = _pad16(s0)
    W0p = jnp.pad(W0, ((0, 0), (0, wp0 - s0)))
    R0p = jnp.pad(R0, ((0, 0), (0, wp0 - s0)))
    b0p = jnp.pad(b0, (0, wp0 - s0)).reshape(1, wp0)
    m0, r0 = pl.pallas_call(
        _proj0_body,
        out_shape=[jax.ShapeDtypeStruct((N_NODES, wp0), jnp.float32),
                   jax.ShapeDtypeStruct((N_NODES, wp0), jnp.float32)],
    )(x, W0p, R0p)
    zeros_tab = jnp.zeros((N_NODES, 16), jnp.float32)
    agg_lo, agg_hi = _make_edge_agg(16)(m0, zeros_tab, srcp, dstp, attrp)
    h_lo = pl.pallas_call(
        _l0post_body,
        out_shape=jax.ShapeDtypeStruct((N_NODES, wp0), jnp.float32),
    )(agg_lo, r0, b0p)
    h_hi = zeros_tab
    wprev = wp0
    whprev = 16

    for W_rel, b_rel, W_root in conv_params[1:]:
        prev, size = W_rel.shape
        wout = _pad16(size)
        wh_out = _half_w(wout)
        Wp = jnp.pad(W_rel, ((0, wprev - prev), (0, wout - size)))
        Rp = jnp.pad(W_root, ((0, wprev - prev), (0, wout - size)))
        bp = jnp.pad(b_rel, (0, wout - size)).reshape(1, wout)
        wrl, wrh = _split_cols(Wp, whprev)
        wtl, wth = _split_cols(Rp, whprev)
        agg_lo, agg_hi = _make_edge_agg(whprev)(h_lo, h_hi, srcp, dstp, attrp)
        h_lo, h_hi = pl.pallas_call(
            functools.partial(_dense_body, wh_out),
            out_shape=[jax.ShapeDtypeStruct((N_NODES, wh_out), jnp.float32),
                       jax.ShapeDtypeStruct((N_NODES, wh_out), jnp.float32)],
        )(agg_lo, agg_hi, h_lo, h_hi, wrl, wrh, wtl, wth, bp)
        wprev = wout
        whprev = wh_out

    wp2 = waypoints.astype(jnp.int32).reshape(NB_WAY, 1)
    cf = 2 * whprev
    fc2w_p = jnp.pad(fc2_W, ((0, 0), (0, 28)))
    fc2b_p = jnp.pad(fc2_b, (0, 28), constant_values=-jnp.inf)
    return pl.pallas_call(
        _final_body,
        out_shape=jax.ShapeDtypeStruct((1, NB_WAY, NB_WAY), jnp.float32),
    )(h_lo, h_hi, wp2, bn_gamma.reshape(1, cf), bn_beta.reshape(1, cf),
      fc1_W, fc1_b.reshape(1, -1), fc2w_p, fc2b_p.reshape(1, -1))


# speedup vs baseline: 10.7865x; 1.4082x over previous
import functools

import jax
import jax.numpy as jnp
from jax import lax
from jax.experimental import pallas as pl
from jax.experimental.pallas import tpu as pltpu
from jax.experimental.pallas import tpu_sc as plsc

N_NODES = 10000
NB_WAY = 64
NSUB = 16
GB = 128
GRP_E = 80
GRP_C = 160
E_PAD = 2 * NSUB * GRP_E * GB
N_ACC = 10240
ROWS_PER_SUB = N_ACC // NSUB
COL_W = 80


def _sigmoid(x):
    return 1.0 / (1.0 + jnp.exp(-x))


def _pad16(n):
    return -(-n // 16) * 16


@functools.lru_cache(maxsize=None)
def _make_edge_agg(wh, grp):
    nj = wh // 16
    if wh >= 112:
        chg = 16
    elif grp % 40 == 0:
        chg = 40
    else:
        chg = 32
    nch = grp // chg
    mesh = plsc.VectorSubcoreMesh(core_axis_name="c", subcore_axis_name="s")

    @functools.partial(
        pl.kernel, mesh=mesh,
        compiler_params=pltpu.CompilerParams(use_tc_tiling_on_sc=False),
        out_type=[jax.ShapeDtypeStruct((N_ACC, wh), jnp.float32),
                  jax.ShapeDtypeStruct((N_ACC, wh), jnp.float32)],
        scratch_types=[
            pltpu.VMEM((chg, GB), jnp.int32),
            pltpu.VMEM((chg, GB), jnp.int32),
            pltpu.VMEM((chg, GB), jnp.float32),
            pltpu.VMEM((GB, wh), jnp.float32),
            pltpu.VMEM((GB, wh), jnp.float32),
            pltpu.VMEM_SHARED((N_ACC, wh), jnp.float32),
            pltpu.SemaphoreType.DMA,
            pltpu.SemaphoreType.DMA,
        ],
    )
    def edge_agg(m0_hbm, m1_hbm, src_hbm, dst_hbm, attr_hbm,
                 out0, out1, src_v, dst_v, attr_v, rows0, rows1,
                 agg_sh, sem0, sem1):
        cid = lax.axis_index("c")
        sid = lax.axis_index("s")

        zero16 = jnp.zeros((16,), jnp.float32)

        def zrow(i, carry):
            for j in range(nj):
                rows0[i, pl.ds(j * 16, 16)] = zero16
            return carry

        lax.fori_loop(0, GB, zrow, 0)
        base = sid * ROWS_PER_SUB
        for q in range(ROWS_PER_SUB // GB):
            pltpu.sync_copy(rows0, agg_sh.at[pl.ds(base + q * GB, GB)])
        plsc.subcore_barrier()

        def scale(rows, g):
            def body(bb, carry):
                av = attr_v[g, pl.ds(bb * 16, 16)]
                for e in range(16):
                    a16 = jnp.full((16,), av[e], jnp.float32)
                    b = bb * 16 + e
                    for j in range(nj):
                        rows[b, pl.ds(j * 16, 16)] = (
                            rows[b, pl.ds(j * 16, 16)] * a16)
                return carry
            lax.fori_loop(0, GB // 16, body, 0)

        def main_loop(m_hbm):
            def chunk(c, carry):
                pltpu.sync_copy(src_hbm.at[cid, sid, pl.ds(c * chg, chg)],
                                src_v)
                pltpu.sync_copy(dst_hbm.at[cid, sid, pl.ds(c * chg, chg)],
                                dst_v)
                pltpu.sync_copy(attr_hbm.at[cid, sid, pl.ds(c * chg, chg)],
                                attr_v)
                pltpu.async_copy(m_hbm.at[src_v.at[0]], rows0, sem0)

                def step(p, carry):
                    g0 = 2 * p
                    g1 = 2 * p + 1
                    pltpu.make_async_copy(
                        m_hbm.at[src_v.at[g0]], rows0, sem0).wait()
                    pltpu.async_copy(m_hbm.at[src_v.at[g1]], rows1, sem1)
                    scale(rows0, g0)
                    pltpu.sync_copy(rows0, agg_sh.at[dst_v.at[g0]], add=True)
                    g2 = jnp.where(g1 + 1 < chg, g1 + 1, 0)
                    pltpu.make_async_copy(
                        m_hbm.at[src_v.at[g1]], rows1, sem1).wait()
                    pltpu.async_copy(m_hbm.at[src_v.at[g2]], rows0, sem0)
                    scale(rows1, g1)
                    pltpu.sync_copy(rows1, agg_sh.at[dst_v.at[g1]], add=True)
                    return carry

                lax.fori_loop(0, chg // 2, step, 0)
                pltpu.make_async_copy(
                    m_hbm.at[src_v.at[0]], rows0, sem0).wait()
                return carry

            lax.fori_loop(0, nch, chunk, 0)

        @pl.when(cid == 0)
        def _():
            main_loop(m0_hbm)

        @pl.when(cid == 1)
        def _():
            main_loop(m1_hbm)

        plsc.subcore_barrier()

        @pl.when(cid == 0)
        def _():
            pltpu.sync_copy(agg_sh.at[pl.ds(base, ROWS_PER_SUB)],
                            out0.at[pl.ds(base, ROWS_PER_SUB)])

        @pl.when(cid == 1)
        def _():
            pltpu.sync_copy(agg_sh.at[pl.ds(base, ROWS_PER_SUB)],
                            out1.at[pl.ds(base, ROWS_PER_SUB)])

    return edge_agg


def _proj0_body(x_ref, wrel_ref, wroot_ref, m_ref, r_ref):
    x = x_ref[...]
    m_ref[...] = jnp.dot(x, wrel_ref[...], preferred_element_type=jnp.float32)
    r_ref[...] = jnp.dot(x, wroot_ref[...], preferred_element_type=jnp.float32)


def _l0post_body(a0_ref, a1_ref, r_ref, b_ref, out_ref):
    out_ref[...] = _sigmoid(
        a0_ref[:N_NODES] + a1_ref[:N_NODES] + r_ref[...] + b_ref[0, :])


def _dense_body(mode_in, split_out, wout, *refs):
    if mode_in == "edge":
        a0_ref, a1_ref, h0_ref, w_ref, r_ref, b_ref = refs[:6]
        out_refs = refs[6:]
        z = jnp.dot(a0_ref[:N_NODES] + a1_ref[:N_NODES], w_ref[...],
                    preferred_element_type=jnp.float32)
        z = z + jnp.dot(h0_ref[...], r_ref[...],
                        preferred_element_type=jnp.float32)
    else:
        a0_ref, a1_ref, h0_ref, h1_ref, w_ref, r_ref, b_ref = refs[:7]
        out_refs = refs[7:]
        w = w_ref[...]
        r = r_ref[...]
        z = (jnp.dot(a0_ref[:N_NODES], w[:COL_W],
                     preferred_element_type=jnp.float32)
             + jnp.dot(a1_ref[:N_NODES], w[COL_W:],
                       preferred_element_type=jnp.float32)
             + jnp.dot(h0_ref[...], r[:COL_W],
                       preferred_element_type=jnp.float32)
             + jnp.dot(h1_ref[...], r[COL_W:],
                       preferred_element_type=jnp.float32))
    h = _sigmoid(z + b_ref[0, :])
    if not split_out:
        out_refs[0][...] = h
    else:
        n = h.shape[0]
        out_refs[0][...] = h[:, :COL_W]
        hi = h[:, COL_W:]
        if wout - COL_W < COL_W:
            hi = jnp.concatenate(
                [hi, jnp.zeros((n, 2 * COL_W - wout), jnp.float32)], axis=1)
        out_refs[1][...] = hi


def _final_body(hlo_ref, hhi_ref, wp_ref, gamma_ref, beta_ref, fc1a_ref,
                fc1b_ref, fc2w_ref, fc2b_ref, out_ref):
    h = jnp.concatenate([hlo_ref[...], hhi_ref[...]], axis=1)
    n = h.shape[0]
    cf = h.shape[1]
    mean = jnp.sum(h, axis=0) / n
    var = jnp.sum(h * h, axis=0) / n - mean * mean
    inv = gamma_ref[0, :] / jnp.sqrt(var + 1e-5)

    wp = wp_ref[...]
    cols = lax.broadcasted_iota(jnp.int32, (NB_WAY, n), 1)
    onehot = jnp.where(cols == wp, 1.0, 0.0).astype(jnp.float32)
    sel = jnp.dot(onehot, h, preferred_element_type=jnp.float32)
    sel = (sel - mean) * inv + beta_ref[0, :]

    fc1 = fc1a_ref[...]
    A = jnp.dot(sel, fc1[:cf], preferred_element_type=jnp.float32)
    A = A + fc1b_ref[0, :]
    B = jnp.dot(sel, fc1[cf:], preferred_element_type=jnp.float32)

    fc2w = fc2w_ref[...]
    fc2b = fc2b_ref[0, :]
    CH = 8
    rows = []
    for c in range(NB_WAY // CH):
        Ac = A[c * CH:(c + 1) * CH]
        hid = Ac[:, None, :] + B[None, :, :]
        hid = _sigmoid(hid).reshape(CH * NB_WAY, A.shape[1])
        p = jnp.dot(hid, fc2w, preferred_element_type=jnp.float32) + fc2b
        rows.append(jnp.max(p, axis=1).reshape(CH, NB_WAY))
    t = _sigmoid(jnp.concatenate(rows, axis=0))
    eye = (lax.broadcasted_iota(jnp.int32, (NB_WAY, NB_WAY), 0)
           == lax.broadcasted_iota(jnp.int32, (NB_WAY, NB_WAY), 1))
    t = jnp.where(eye, 0.0, t)
    out_ref[0, :, :] = t.T


def kernel(x, edge_index, edge_attr, waypoints, conv_params, bn_gamma,
           bn_beta, fc1_W, fc1_b, fc2_W, fc2_b):
    src = edge_index[0].astype(jnp.int32)
    dst = edge_index[1].astype(jnp.int32)
    npad = E_PAD - src.shape[0]
    pad_idx = (jnp.arange(npad, dtype=jnp.int32) * 97) % N_NODES
    srcp = jnp.concatenate([src, pad_idx])
    dstp = jnp.concatenate([dst, pad_idx])
    attrp = jnp.concatenate([edge_attr, jnp.zeros((npad,), jnp.float32)])
    edges_e = tuple(a.reshape(2, NSUB, GRP_E, GB) for a in (srcp, dstp, attrp))
    edges_c = tuple(
        jnp.broadcast_to(a.reshape(1, NSUB, GRP_C, GB), (2, NSUB, GRP_C, GB))
        for a in (srcp, dstp, attrp))

    W0, b0, R0 = conv_params[0]
    s0 = W0.shape[1]
    wp0 = _pad16(s0)
    W0p = jnp.pad(W0, ((0, 0), (0, wp0 - s0)))
    R0p = jnp.pad(R0, ((0, 0), (0, wp0 - s0)))
    b0p = jnp.pad(b0, (0, wp0 - s0)).reshape(1, wp0)
    m0, r0 = pl.pallas_call(
        _proj0_body,
        out_shape=[jax.ShapeDtypeStruct((N_NODES, wp0), jnp.float32),
                   jax.ShapeDtypeStruct((N_NODES, wp0), jnp.float32)],
    )(x, W0p, R0p)
    a0, a1 = _make_edge_agg(wp0, GRP_E)(m0, m0, *edges_e)
    h = pl.pallas_call(
        _l0post_body,
        out_shape=jax.ShapeDtypeStruct((N_NODES, wp0), jnp.float32),
    )(a0, a1, r0, b0p)
    h_parts = (h,)
    wprev = wp0

    nlayers = len(conv_params)
    for i in range(1, nlayers):
        W_rel, b_rel, W_root = conv_params[i]
        prev, size = W_rel.shape
        wout = _pad16(size)
        mode_in = "edge" if wprev <= 128 else "col"
        split_out = (i < nlayers - 1 and wout > 128) or i == nlayers - 1
        win = wprev if mode_in == "edge" else 2 * COL_W
        Wp = jnp.pad(W_rel, ((0, win - prev), (0, wout - size)))
        Rp = jnp.pad(W_root, ((0, win - prev), (0, wout - size)))
        bp = jnp.pad(b_rel, (0, wout - size)).reshape(1, wout)
        if mode_in == "edge":
            a0, a1 = _make_edge_agg(wprev, GRP_E)(
                h_parts[0], h_parts[0], *edges_e)
        else:
            a0, a1 = _make_edge_agg(COL_W, GRP_C)(
                h_parts[0], h_parts[1], *edges_c)
        if split_out:
            out_shape = [
                jax.ShapeDtypeStruct((N_NODES, COL_W), jnp.float32),
                jax.ShapeDtypeStruct((N_NODES, COL_W), jnp.float32)]
        else:
            out_shape = [jax.ShapeDtypeStruct((N_NODES, wout), jnp.float32)]
        h_parts = pl.pallas_call(
            functools.partial(_dense_body, mode_in, split_out, wout),
            out_shape=out_shape,
        )(a0, a1, *h_parts, Wp, Rp, bp)
        h_parts = tuple(h_parts)
        wprev = wout

    wp2 = waypoints.astype(jnp.int32).reshape(NB_WAY, 1)
    cf = 2 * COL_W
    fc2w_p = jnp.pad(fc2_W, ((0, 0), (0, 28)))
    fc2b_p = jnp.pad(fc2_b, (0, 28), constant_values=-jnp.inf)
    return pl.pallas_call(
        _final_body,
        out_shape=jax.ShapeDtypeStruct((1, NB_WAY, NB_WAY), jnp.float32),
    )(h_parts[0], h_parts[1], wp2, bn_gamma.reshape(1, cf),
      bn_beta.reshape(1, cf), fc1_W, fc1_b.reshape(1, -1), fc2w_p,
      fc2b_p.reshape(1, -1))


# speedup vs baseline: 10.8219x; 1.0033x over previous
import functools

import jax
import jax.numpy as jnp
from jax import lax
from jax.experimental import pallas as pl
from jax.experimental.pallas import tpu as pltpu
from jax.experimental.pallas import tpu_sc as plsc

N_NODES = 10000
NB_WAY = 64
NSUB = 16
GB = 128
GRP_E = 80
GRP_C = 160
E_PAD = 2 * NSUB * GRP_E * GB
N_ACC = 10240
ROWS_PER_SUB = N_ACC // NSUB
COL_W = 80


def _sigmoid(x):
    return 1.0 / (1.0 + jnp.exp(-x))


def _pad16(n):
    return -(-n // 16) * 16


@functools.lru_cache(maxsize=None)
def _make_edge_agg(wh, grp):
    nj = wh // 16
    if wh >= 112:
        chg = 16
    elif grp % 40 == 0:
        chg = 40
    else:
        chg = 32
    nch = grp // chg
    mesh = plsc.VectorSubcoreMesh(core_axis_name="c", subcore_axis_name="s")

    @functools.partial(
        pl.kernel, mesh=mesh,
        compiler_params=pltpu.CompilerParams(use_tc_tiling_on_sc=False),
        out_type=[jax.ShapeDtypeStruct((N_ACC, wh), jnp.float32),
                  jax.ShapeDtypeStruct((N_ACC, wh), jnp.float32)],
        scratch_types=[
            pltpu.VMEM((chg, GB), jnp.int32),
            pltpu.VMEM((chg, GB), jnp.int32),
            pltpu.VMEM((chg, GB), jnp.float32),
            pltpu.VMEM((GB, wh), jnp.float32),
            pltpu.VMEM((GB, wh), jnp.float32),
            pltpu.VMEM_SHARED((N_ACC, wh), jnp.float32),
            pltpu.SemaphoreType.DMA,
            pltpu.SemaphoreType.DMA,
            pltpu.SemaphoreType.DMA,
            pltpu.SemaphoreType.DMA,
        ],
    )
    def edge_agg(m0_hbm, m1_hbm, src_hbm, dst_hbm, attr_hbm,
                 out0, out1, src_v, dst_v, attr_v, rows0, rows1,
                 agg_sh, sem0, sem1, sem2, sem3):
        cid = lax.axis_index("c")
        sid = lax.axis_index("s")

        zero16 = jnp.zeros((16,), jnp.float32)

        def zrow(i, carry):
            for j in range(nj):
                rows0[i, pl.ds(j * 16, 16)] = zero16
            return carry

        lax.fori_loop(0, GB, zrow, 0)
        base = sid * ROWS_PER_SUB
        for q in range(ROWS_PER_SUB // GB):
            pltpu.sync_copy(rows0, agg_sh.at[pl.ds(base + q * GB, GB)])
        plsc.subcore_barrier()

        def scale(rows, g):
            def body(bb, carry):
                av = attr_v[g, pl.ds(bb * 16, 16)]
                for e in range(16):
                    a16 = jnp.full((16,), av[e], jnp.float32)
                    b = bb * 16 + e
                    for j in range(nj):
                        rows[b, pl.ds(j * 16, 16)] = (
                            rows[b, pl.ds(j * 16, 16)] * a16)
                return carry
            lax.fori_loop(0, GB // 16, body, 0)

        def main_loop(m_hbm):
            def chunk(c, carry):
                pltpu.sync_copy(src_hbm.at[cid, sid, pl.ds(c * chg, chg)],
                                src_v)
                pltpu.sync_copy(dst_hbm.at[cid, sid, pl.ds(c * chg, chg)],
                                dst_v)
                pltpu.sync_copy(attr_hbm.at[cid, sid, pl.ds(c * chg, chg)],
                                attr_v)
                pltpu.async_copy(m_hbm.at[src_v.at[0]], rows0, sem0)

                def step(p, carry):
                    g0 = 2 * p
                    g1 = 2 * p + 1
                    pltpu.make_async_copy(
                        m_hbm.at[src_v.at[g0]], rows0, sem0).wait()

                    @pl.when(p > 0)
                    def _():
                        pltpu.make_async_copy(
                            rows1, agg_sh.at[dst_v.at[g0]], sem3).wait()

                    pltpu.async_copy(m_hbm.at[src_v.at[g1]], rows1, sem1)
                    scale(rows0, g0)
                    pltpu.async_copy(rows0, agg_sh.at[dst_v.at[g0]], sem2,
                                     add=True)
                    g2 = jnp.where(g1 + 1 < chg, g1 + 1, 0)
                    pltpu.make_async_copy(
                        m_hbm.at[src_v.at[g1]], rows1, sem1).wait()
                    pltpu.make_async_copy(
                        rows0, agg_sh.at[dst_v.at[g0]], sem2).wait()
                    pltpu.async_copy(m_hbm.at[src_v.at[g2]], rows0, sem0)
                    scale(rows1, g1)
                    pltpu.async_copy(rows1, agg_sh.at[dst_v.at[g1]], sem3,
                                     add=True)
                    return carry

                lax.fori_loop(0, chg // 2, step, 0)
                pltpu.make_async_copy(
                    m_hbm.at[src_v.at[0]], rows0, sem0).wait()
                pltpu.make_async_copy(
                    rows1, agg_sh.at[dst_v.at[0]], sem3).wait()
                return carry

            lax.fori_loop(0, nch, chunk, 0)

        @pl.when(cid == 0)
        def _():
            main_loop(m0_hbm)

        @pl.when(cid == 1)
        def _():
            main_loop(m1_hbm)

        plsc.subcore_barrier()

        @pl.when(cid == 0)
        def _():
            pltpu.sync_copy(agg_sh.at[pl.ds(base, ROWS_PER_SUB)],
                            out0.at[pl.ds(base, ROWS_PER_SUB)])

        @pl.when(cid == 1)
        def _():
            pltpu.sync_copy(agg_sh.at[pl.ds(base, ROWS_PER_SUB)],
                            out1.at[pl.ds(base, ROWS_PER_SUB)])

    return edge_agg


def _proj0_body(x_ref, wrel_ref, wroot_ref, m_ref, r_ref):
    x = x_ref[...]
    m_ref[...] = jnp.dot(x, wrel_ref[...], preferred_element_type=jnp.float32)
    r_ref[...] = jnp.dot(x, wroot_ref[...], preferred_element_type=jnp.float32)


def _l0post_body(a0_ref, a1_ref, r_ref, b_ref, out_ref):
    out_ref[...] = _sigmoid(
        a0_ref[:N_NODES] + a1_ref[:N_NODES] + r_ref[...] + b_ref[0, :])


def _dense_body(mode_in, split_out, wout, *refs):
    if mode_in == "edge":
        a0_ref, a1_ref, h0_ref, w_ref, r_ref, b_ref = refs[:6]
        out_refs = refs[6:]
        z = jnp.dot(a0_ref[:N_NODES] + a1_ref[:N_NODES], w_ref[...],
                    preferred_element_type=jnp.float32)
        z = z + jnp.dot(h0_ref[...], r_ref[...],
                        preferred_element_type=jnp.float32)
    else:
        a0_ref, a1_ref, h0_ref, h1_ref, w_ref, r_ref, b_ref = refs[:7]
        out_refs = refs[7:]
        w = w_ref[...]
        r = r_ref[...]
        z = (jnp.dot(a0_ref[:N_NODES], w[:COL_W],
                     preferred_element_type=jnp.float32)
             + jnp.dot(a1_ref[:N_NODES], w[COL_W:],
                       preferred_element_type=jnp.float32)
             + jnp.dot(h0_ref[...], r[:COL_W],
                       preferred_element_type=jnp.float32)
             + jnp.dot(h1_ref[...], r[COL_W:],
                       preferred_element_type=jnp.float32))
    h = _sigmoid(z + b_ref[0, :])
    if not split_out:
        out_refs[0][...] = h
    else:
        n = h.shape[0]
        out_refs[0][...] = h[:, :COL_W]
        hi = h[:, COL_W:]
        if wout - COL_W < COL_W:
            hi = jnp.concatenate(
                [hi, jnp.zeros((n, 2 * COL_W - wout), jnp.float32)], axis=1)
        out_refs[1][...] = hi


def _final_body(hlo_ref, hhi_ref, wp_ref, gamma_ref, beta_ref, fc1a_ref,
                fc1b_ref, fc2w_ref, fc2b_ref, out_ref):
    h = jnp.concatenate([hlo_ref[...], hhi_ref[...]], axis=1)
    n = h.shape[0]
    cf = h.shape[1]
    mean = jnp.sum(h, axis=0) / n
    var = jnp.sum(h * h, axis=0) / n - mean * mean
    inv = gamma_ref[0, :] / jnp.sqrt(var + 1e-5)

    wp = wp_ref[...]
    cols = lax.broadcasted_iota(jnp.int32, (NB_WAY, n), 1)
    onehot = jnp.where(cols == wp, 1.0, 0.0).astype(jnp.float32)
    sel = jnp.dot(onehot, h, preferred_element_type=jnp.float32)
    sel = (sel - mean) * inv + beta_ref[0, :]

    fc1 = fc1a_ref[...]
    A = jnp.dot(sel, fc1[:cf], preferred_element_type=jnp.float32)
    A = A + fc1b_ref[0, :]
    B = jnp.dot(sel, fc1[cf:], preferred_element_type=jnp.float32)

    fc2w = fc2w_ref[...]
    fc2b = fc2b_ref[0, :]
    CH = 8
    rows = []
    for c in range(NB_WAY // CH):
        Ac = A[c * CH:(c + 1) * CH]
        hid = Ac[:, None, :] + B[None, :, :]
        hid = _sigmoid(hid).reshape(CH * NB_WAY, A.shape[1])
        p = jnp.dot(hid, fc2w, preferred_element_type=jnp.float32) + fc2b
        rows.append(jnp.max(p, axis=1).reshape(CH, NB_WAY))
    t = _sigmoid(jnp.concatenate(rows, axis=0))
    eye = (lax.broadcasted_iota(jnp.int32, (NB_WAY, NB_WAY), 0)
           == lax.broadcasted_iota(jnp.int32, (NB_WAY, NB_WAY), 1))
    t = jnp.where(eye, 0.0, t)
    out_ref[0, :, :] = t.T


def kernel(x, edge_index, edge_attr, waypoints, conv_params, bn_gamma,
           bn_beta, fc1_W, fc1_b, fc2_W, fc2_b):
    src = edge_index[0].astype(jnp.int32)
    dst = edge_index[1].astype(jnp.int32)
    npad = E_PAD - src.shape[0]
    pad_idx = (jnp.arange(npad, dtype=jnp.int32) * 97) % N_NODES
    srcp = jnp.concatenate([src, pad_idx])
    dstp = jnp.concatenate([dst, pad_idx])
    attrp = jnp.concatenate([edge_attr, jnp.zeros((npad,), jnp.float32)])
    edges_e = tuple(a.reshape(2, NSUB, GRP_E, GB) for a in (srcp, dstp, attrp))
    edges_c = tuple(
        jnp.broadcast_to(a.reshape(1, NSUB, GRP_C, GB), (2, NSUB, GRP_C, GB))
        for a in (srcp, dstp, attrp))

    W0, b0, R0 = conv_params[0]
    s0 = W0.shape[1]
    wp0 = _pad16(s0)
    W0p = jnp.pad(W0, ((0, 0), (0, wp0 - s0)))
    R0p = jnp.pad(R0, ((0, 0), (0, wp0 - s0)))
    b0p = jnp.pad(b0, (0, wp0 - s0)).reshape(1, wp0)
    m0, r0 = pl.pallas_call(
        _proj0_body,
        out_shape=[jax.ShapeDtypeStruct((N_NODES, wp0), jnp.float32),
                   jax.ShapeDtypeStruct((N_NODES, wp0), jnp.float32)],
    )(x, W0p, R0p)
    a0, a1 = _make_edge_agg(wp0, GRP_E)(m0, m0, *edges_e)
    h = pl.pallas_call(
        _l0post_body,
        out_shape=jax.ShapeDtypeStruct((N_NODES, wp0), jnp.float32),
    )(a0, a1, r0, b0p)
    h_parts = (h,)
    wprev = wp0

    nlayers = len(conv_params)
    for i in range(1, nlayers):
        W_rel, b_rel, W_root = conv_params[i]
        prev, size = W_rel.shape
        wout = _pad16(size)
        mode_in = "edge" if wprev <= 128 else "col"
        split_out = (i < nlayers - 1 and wout > 128) or i == nlayers - 1
        win = wprev if mode_in == "edge" else 2 * COL_W
        Wp = jnp.pad(W_rel, ((0, win - prev), (0, wout - size)))
        Rp = jnp.pad(W_root, ((0, win - prev), (0, wout - size)))
        bp = jnp.pad(b_rel, (0, wout - size)).reshape(1, wout)
        if mode_in == "edge":
            a0, a1 = _make_edge_agg(wprev, GRP_E)(
                h_parts[0], h_parts[0], *edges_e)
        else:
            a0, a1 = _make_edge_agg(COL_W, GRP_C)(
                h_parts[0], h_parts[1], *edges_c)
        if split_out:
            out_shape = [
                jax.ShapeDtypeStruct((N_NODES, COL_W), jnp.float32),
                jax.ShapeDtypeStruct((N_NODES, COL_W), jnp.float32)]
        else:
            out_shape = [jax.ShapeDtypeStruct((N_NODES, wout), jnp.float32)]
        h_parts = pl.pallas_call(
            functools.partial(_dense_body, mode_in, split_out, wout),
            out_shape=out_shape,
        )(a0, a1, *h_parts, Wp, Rp, bp)
        h_parts = tuple(h_parts)
        wprev = wout

    wp2 = waypoints.astype(jnp.int32).reshape(NB_WAY, 1)
    cf = 2 * COL_W
    fc2w_p = jnp.pad(fc2_W, ((0, 0), (0, 28)))
    fc2b_p = jnp.pad(fc2_b, (0, 28), constant_values=-jnp.inf)
    return pl.pallas_call(
        _final_body,
        out_shape=jax.ShapeDtypeStruct((1, NB_WAY, NB_WAY), jnp.float32),
    )(h_parts[0], h_parts[1], wp2, bn_gamma.reshape(1, cf),
      bn_beta.reshape(1, cf), fc1_W, fc1_b.reshape(1, -1), fc2w_p,
      fc2b_p.reshape(1, -1))


# speedup vs baseline: 11.9594x; 1.1051x over previous
import functools

import jax
import jax.numpy as jnp
from jax import lax
from jax.experimental import pallas as pl
from jax.experimental.pallas import tpu as pltpu
from jax.experimental.pallas import tpu_sc as plsc

N_NODES = 10000
NB_WAY = 64
NSUB = 16
GB = 128
GRP_E = 80
GRP_C = 160
E_PAD = 2 * NSUB * GRP_E * GB
N_ACC = 10240
ROWS_PER_SUB = N_ACC // NSUB
COL_W = 80


def _sigmoid(x):
    return 1.0 / (1.0 + jnp.exp(-x))


def _pad16(n):
    return -(-n // 16) * 16


@functools.lru_cache(maxsize=None)
def _make_edge_agg(wh, grp):
    nj = wh // 16
    if wh >= 112:
        chg = 16
    elif grp % 40 == 0:
        chg = 40
    else:
        chg = 32
    nch = grp // chg
    if wh <= 48:
        nbuf = 8
    elif wh <= 80:
        nbuf = 4
    else:
        nbuf = 2
    assert chg % nbuf == 0
    mesh = plsc.VectorSubcoreMesh(core_axis_name="c", subcore_axis_name="s")

    @functools.partial(
        pl.kernel, mesh=mesh,
        compiler_params=pltpu.CompilerParams(use_tc_tiling_on_sc=False),
        out_type=[jax.ShapeDtypeStruct((N_ACC, wh), jnp.float32),
                  jax.ShapeDtypeStruct((N_ACC, wh), jnp.float32)],
        scratch_types=(
            [pltpu.VMEM((chg, GB), jnp.int32),
             pltpu.VMEM((chg, GB), jnp.int32),
             pltpu.VMEM((chg, GB), jnp.float32)]
            + [pltpu.VMEM((GB, wh), jnp.float32) for _ in range(nbuf)]
            + [pltpu.VMEM_SHARED((N_ACC, wh), jnp.float32)]
            + [pltpu.SemaphoreType.DMA for _ in range(2 * nbuf)]
        ),
    )
    def edge_agg(m0_hbm, m1_hbm, src_hbm, dst_hbm, attr_hbm,
                 out0, out1, src_v, dst_v, attr_v, *rest):
        rows = rest[:nbuf]
        agg_sh = rest[nbuf]
        sem_g = rest[nbuf + 1:2 * nbuf + 1]
        sem_s = rest[2 * nbuf + 1:]
        cid = lax.axis_index("c")
        sid = lax.axis_index("s")

        zero16 = jnp.zeros((16,), jnp.float32)

        def zrow(i, carry):
            for j in range(nj):
                rows[0][i, pl.ds(j * 16, 16)] = zero16
            return carry

        lax.fori_loop(0, GB, zrow, 0)
        base = sid * ROWS_PER_SUB
        for q in range(ROWS_PER_SUB // GB):
            pltpu.sync_copy(rows[0], agg_sh.at[pl.ds(base + q * GB, GB)])
        plsc.subcore_barrier()

        def scale(rws, g):
            def body(bb, carry):
                av = attr_v[g, pl.ds(bb * 16, 16)]
                for e in range(16):
                    a16 = jnp.full((16,), av[e], jnp.float32)
                    b = bb * 16 + e
                    for j in range(nj):
                        rws[b, pl.ds(j * 16, 16)] = (
                            rws[b, pl.ds(j * 16, 16)] * a16)
                return carry
            lax.fori_loop(0, GB // 16, body, 0)

        def main_loop(m_hbm):
            def chunk(c, carry):
                pltpu.sync_copy(src_hbm.at[cid, sid, pl.ds(c * chg, chg)],
                                src_v)
                pltpu.sync_copy(dst_hbm.at[cid, sid, pl.ds(c * chg, chg)],
                                dst_v)
                pltpu.sync_copy(attr_hbm.at[cid, sid, pl.ds(c * chg, chg)],
                                attr_v)
                for b in range(nbuf - 1):
                    pltpu.async_copy(m_hbm.at[src_v.at[b]], rows[b],
                                     sem_g[b])

                def qstep(q, carry):
                    for b in range(nbuf):
                        g = q * nbuf + b
                        b2 = (b - 1) % nbuf
                        pltpu.make_async_copy(
                            m_hbm.at[src_v.at[g]], rows[b], sem_g[b]).wait()
                        scale(rows[b], g)
                        pltpu.async_copy(rows[b], agg_sh.at[dst_v.at[g]],
                                         sem_s[b], add=True)
                        if b == 0:
                            @pl.when(q > 0)
                            def _():
                                pltpu.make_async_copy(
                                    rows[b2], agg_sh.at[dst_v.at[g]],
                                    sem_s[b2]).wait()
                        else:
                            pltpu.make_async_copy(
                                rows[b2], agg_sh.at[dst_v.at[g]],
                                sem_s[b2]).wait()
                        ga = jnp.where(g + nbuf - 1 < chg, g + nbuf - 1, 0)
                        pltpu.async_copy(m_hbm.at[src_v.at[ga]], rows[b2],
                                         sem_g[b2])
                    return carry

                lax.fori_loop(0, chg // nbuf, qstep, 0)
                for b in range(nbuf - 1):
                    pltpu.make_async_copy(
                        m_hbm.at[src_v.at[0]], rows[b], sem_g[b]).wait()
                pltpu.make_async_copy(
                    rows[nbuf - 1], agg_sh.at[dst_v.at[0]],
                    sem_s[nbuf - 1]).wait()
                return carry

            lax.fori_loop(0, nch, chunk, 0)

        @pl.when(cid == 0)
        def _():
            main_loop(m0_hbm)

        @pl.when(cid == 1)
        def _():
            main_loop(m1_hbm)

        plsc.subcore_barrier()

        @pl.when(cid == 0)
        def _():
            pltpu.sync_copy(agg_sh.at[pl.ds(base, ROWS_PER_SUB)],
                            out0.at[pl.ds(base, ROWS_PER_SUB)])

        @pl.when(cid == 1)
        def _():
            pltpu.sync_copy(agg_sh.at[pl.ds(base, ROWS_PER_SUB)],
                            out1.at[pl.ds(base, ROWS_PER_SUB)])

    return edge_agg


def _proj0_body(x_ref, wrel_ref, wroot_ref, m_ref, r_ref):
    x = x_ref[...]
    m_ref[...] = jnp.dot(x, wrel_ref[...], preferred_element_type=jnp.float32)
    r_ref[...] = jnp.dot(x, wroot_ref[...], preferred_element_type=jnp.float32)


def _l0post_body(a0_ref, a1_ref, r_ref, b_ref, out_ref):
    out_ref[...] = _sigmoid(
        a0_ref[:N_NODES] + a1_ref[:N_NODES] + r_ref[...] + b_ref[0, :])


def _dense_body(mode_in, split_out, wout, *refs):
    if mode_in == "edge":
        a0_ref, a1_ref, h0_ref, w_ref, r_ref, b_ref = refs[:6]
        out_refs = refs[6:]
        z = jnp.dot(a0_ref[:N_NODES] + a1_ref[:N_NODES], w_ref[...],
                    preferred_element_type=jnp.float32)
        z = z + jnp.dot(h0_ref[...], r_ref[...],
                        preferred_element_type=jnp.float32)
    else:
        a0_ref, a1_ref, h0_ref, h1_ref, w_ref, r_ref, b_ref = refs[:7]
        out_refs = refs[7:]
        w = w_ref[...]
        r = r_ref[...]
        z = (jnp.dot(a0_ref[:N_NODES], w[:COL_W],
                     preferred_element_type=jnp.float32)
             + jnp.dot(a1_ref[:N_NODES], w[COL_W:],
                       preferred_element_type=jnp.float32)
             + jnp.dot(h0_ref[...], r[:COL_W],
                       preferred_element_type=jnp.float32)
             + jnp.dot(h1_ref[...], r[COL_W:],
                       preferred_element_type=jnp.float32))
    h = _sigmoid(z + b_ref[0, :])
    if not split_out:
        out_refs[0][...] = h
    else:
        n = h.shape[0]
        out_refs[0][...] = h[:, :COL_W]
        hi = h[:, COL_W:]
        if wout - COL_W < COL_W:
            hi = jnp.concatenate(
                [hi, jnp.zeros((n, 2 * COL_W - wout), jnp.float32)], axis=1)
        out_refs[1][...] = hi


def _final_body(hlo_ref, hhi_ref, wp_ref, gamma_ref, beta_ref, fc1a_ref,
                fc1b_ref, fc2w_ref, fc2b_ref, out_ref):
    h = jnp.concatenate([hlo_ref[...], hhi_ref[...]], axis=1)
    n = h.shape[0]
    cf = h.shape[1]
    mean = jnp.sum(h, axis=0) / n
    var = jnp.sum(h * h, axis=0) / n - mean * mean
    inv = gamma_ref[0, :] / jnp.sqrt(var + 1e-5)

    wp = wp_ref[...]
    cols = lax.broadcasted_iota(jnp.int32, (NB_WAY, n), 1)
    onehot = jnp.where(cols == wp, 1.0, 0.0).astype(jnp.float32)
    sel = jnp.dot(onehot, h, preferred_element_type=jnp.float32)
    sel = (sel - mean) * inv + beta_ref[0, :]

    fc1 = fc1a_ref[...]
    A = jnp.dot(sel, fc1[:cf], preferred_element_type=jnp.float32)
    A = A + fc1b_ref[0, :]
    B = jnp.dot(sel, fc1[cf:], preferred_element_type=jnp.float32)

    fc2w = fc2w_ref[...]
    fc2b = fc2b_ref[0, :]
    CH = 8
    rows = []
    for c in range(NB_WAY // CH):
        Ac = A[c * CH:(c + 1) * CH]
        hid = Ac[:, None, :] + B[None, :, :]
        hid = _sigmoid(hid).reshape(CH * NB_WAY, A.shape[1])
        p = jnp.dot(hid, fc2w, preferred_element_type=jnp.float32) + fc2b
        rows.append(jnp.max(p, axis=1).reshape(CH, NB_WAY))
    t = _sigmoid(jnp.concatenate(rows, axis=0))
    eye = (lax.broadcasted_iota(jnp.int32, (NB_WAY, NB_WAY), 0)
           == lax.broadcasted_iota(jnp.int32, (NB_WAY, NB_WAY), 1))
    t = jnp.where(eye, 0.0, t)
    out_ref[0, :, :] = t.T


def kernel(x, edge_index, edge_attr, waypoints, conv_params, bn_gamma,
           bn_beta, fc1_W, fc1_b, fc2_W, fc2_b):
    src = edge_index[0].astype(jnp.int32)
    dst = edge_index[1].astype(jnp.int32)
    npad = E_PAD - src.shape[0]
    pad_idx = (jnp.arange(npad, dtype=jnp.int32) * 97) % N_NODES
    srcp = jnp.concatenate([src, pad_idx])
    dstp = jnp.concatenate([dst, pad_idx])
    attrp = jnp.concatenate([edge_attr, jnp.zeros((npad,), jnp.float32)])
    edges_e = tuple(a.reshape(2, NSUB, GRP_E, GB) for a in (srcp, dstp, attrp))
    edges_c = tuple(
        jnp.broadcast_to(a.reshape(1, NSUB, GRP_C, GB), (2, NSUB, GRP_C, GB))
        for a in (srcp, dstp, attrp))

    W0, b0, R0 = conv_params[0]
    s0 = W0.shape[1]
    wp0 = _pad16(s0)
    W0p = jnp.pad(W0, ((0, 0), (0, wp0 - s0)))
    R0p = jnp.pad(R0, ((0, 0), (0, wp0 - s0)))
    b0p = jnp.pad(b0, (0, wp0 - s0)).reshape(1, wp0)
    m0, r0 = pl.pallas_call(
        _proj0_body,
        out_shape=[jax.ShapeDtypeStruct((N_NODES, wp0), jnp.float32),
                   jax.ShapeDtypeStruct((N_NODES, wp0), jnp.float32)],
    )(x, W0p, R0p)
    a0, a1 = _make_edge_agg(wp0, GRP_E)(m0, m0, *edges_e)
    h = pl.pallas_call(
        _l0post_body,
        out_shape=jax.ShapeDtypeStruct((N_NODES, wp0), jnp.float32),
    )(a0, a1, r0, b0p)
    h_parts = (h,)
    wprev = wp0

    nlayers = len(conv_params)
    for i in range(1, nlayers):
        W_rel, b_rel, W_root = conv_params[i]
        prev, size = W_rel.shape
        wout = _pad16(size)
        mode_in = "edge" if wprev <= 128 else "col"
        split_out = (i < nlayers - 1 and wout > 128) or i == nlayers - 1
        win = wprev if mode_in == "edge" else 2 * COL_W
        Wp = jnp.pad(W_rel, ((0, win - prev), (0, wout - size)))
        Rp = jnp.pad(W_root, ((0, win - prev), (0, wout - size)))
        bp = jnp.pad(b_rel, (0, wout - size)).reshape(1, wout)
        if mode_in == "edge":
            a0, a1 = _make_edge_agg(wprev, GRP_E)(
                h_parts[0], h_parts[0], *edges_e)
        else:
            a0, a1 = _make_edge_agg(COL_W, GRP_C)(
                h_parts[0], h_parts[1], *edges_c)
        if split_out:
            out_shape = [
                jax.ShapeDtypeStruct((N_NODES, COL_W), jnp.float32),
                jax.ShapeDtypeStruct((N_NODES, COL_W), jnp.float32)]
        else:
            out_shape = [jax.ShapeDtypeStruct((N_NODES, wout), jnp.float32)]
        h_parts = pl.pallas_call(
            functools.partial(_dense_body, mode_in, split_out, wout),
            out_shape=out_shape,
        )(a0, a1, *h_parts, Wp, Rp, bp)
        h_parts = tuple(h_parts)
        wprev = wout

    wp2 = waypoints.astype(jnp.int32).reshape(NB_WAY, 1)
    cf = 2 * COL_W
    fc2w_p = jnp.pad(fc2_W, ((0, 0), (0, 28)))
    fc2b_p = jnp.pad(fc2_b, (0, 28), constant_values=-jnp.inf)
    return pl.pallas_call(
        _final_body,
        out_shape=jax.ShapeDtypeStruct((1, NB_WAY, NB_WAY), jnp.float32),
    )(h_parts[0], h_parts[1], wp2, bn_gamma.reshape(1, cf),
      bn_beta.reshape(1, cf), fc1_W, fc1_b.reshape(1, -1), fc2w_p,
      fc2b_p.reshape(1, -1))


# speedup vs baseline: 12.0053x; 1.0038x over previous
import functools

import jax
import jax.numpy as jnp
from jax import lax
from jax.experimental import pallas as pl
from jax.experimental.pallas import tpu as pltpu
from jax.experimental.pallas import tpu_sc as plsc

N_NODES = 10000
NB_WAY = 64
NSUB = 16
GB = 128
GRP_E = 80
GRP_C = 160
E_PAD = 2 * NSUB * GRP_E * GB
N_ACC = 10240
ROWS_PER_SUB = N_ACC // NSUB
COL_W = 80


def _sigmoid(x):
    return 1.0 / (1.0 + jnp.exp(-x))


def _pad16(n):
    return -(-n // 16) * 16


@functools.lru_cache(maxsize=None)
def _make_edge_agg(wh, grp):
    nj = wh // 16
    if wh >= 112:
        chg = 16
    elif wh <= 48 and grp % 80 == 0:
        chg = 80
    elif grp % 40 == 0:
        chg = 40
    else:
        chg = 32
    nch = grp // chg
    if wh <= 64:
        nbuf = 8
    elif wh <= 80:
        nbuf = 4
    else:
        nbuf = 2
    assert chg % nbuf == 0
    mesh = plsc.VectorSubcoreMesh(core_axis_name="c", subcore_axis_name="s")

    @functools.partial(
        pl.kernel, mesh=mesh,
        compiler_params=pltpu.CompilerParams(use_tc_tiling_on_sc=False),
        out_type=[jax.ShapeDtypeStruct((N_ACC, wh), jnp.float32),
                  jax.ShapeDtypeStruct((N_ACC, wh), jnp.float32)],
        scratch_types=(
            [pltpu.VMEM((chg, GB), jnp.int32),
             pltpu.VMEM((chg, GB), jnp.int32),
             pltpu.VMEM((chg, GB), jnp.float32)]
            + [pltpu.VMEM((GB, wh), jnp.float32) for _ in range(nbuf)]
            + [pltpu.VMEM_SHARED((N_ACC, wh), jnp.float32)]
            + [pltpu.SemaphoreType.DMA for _ in range(2 * nbuf)]
        ),
    )
    def edge_agg(m0_hbm, m1_hbm, src_hbm, dst_hbm, attr_hbm,
                 out0, out1, src_v, dst_v, attr_v, *rest):
        rows = rest[:nbuf]
        agg_sh = rest[nbuf]
        sem_g = rest[nbuf + 1:2 * nbuf + 1]
        sem_s = rest[2 * nbuf + 1:]
        cid = lax.axis_index("c")
        sid = lax.axis_index("s")

        zero16 = jnp.zeros((16,), jnp.float32)

        def zrow(i, carry):
            for j in range(nj):
                rows[0][i, pl.ds(j * 16, 16)] = zero16
            return carry

        lax.fori_loop(0, GB, zrow, 0)
        base = sid * ROWS_PER_SUB
        for q in range(ROWS_PER_SUB // GB):
            pltpu.sync_copy(rows[0], agg_sh.at[pl.ds(base + q * GB, GB)])
        plsc.subcore_barrier()

        def scale(rws, g):
            def body(bb, carry):
                av = attr_v[g, pl.ds(bb * 16, 16)]
                for e in range(16):
                    a16 = jnp.full((16,), av[e], jnp.float32)
                    b = bb * 16 + e
                    for j in range(nj):
                        rws[b, pl.ds(j * 16, 16)] = (
                            rws[b, pl.ds(j * 16, 16)] * a16)
                return carry
            lax.fori_loop(0, GB // 16, body, 0)

        def main_loop(m_hbm):
            def chunk(c, carry):
                pltpu.sync_copy(src_hbm.at[cid, sid, pl.ds(c * chg, chg)],
                                src_v)
                pltpu.sync_copy(dst_hbm.at[cid, sid, pl.ds(c * chg, chg)],
                                dst_v)
                pltpu.sync_copy(attr_hbm.at[cid, sid, pl.ds(c * chg, chg)],
                                attr_v)
                for b in range(nbuf - 1):
                    pltpu.async_copy(m_hbm.at[src_v.at[b]], rows[b],
                                     sem_g[b])

                def qstep(q, carry):
                    for b in range(nbuf):
                        g = q * nbuf + b
                        b2 = (b - 1) % nbuf
                        pltpu.make_async_copy(
                            m_hbm.at[src_v.at[g]], rows[b], sem_g[b]).wait()
                        scale(rows[b], g)
                        pltpu.async_copy(rows[b], agg_sh.at[dst_v.at[g]],
                                         sem_s[b], add=True)
                        if b == 0:
                            @pl.when(q > 0)
                            def _():
                                pltpu.make_async_copy(
                                    rows[b2], agg_sh.at[dst_v.at[g]],
                                    sem_s[b2]).wait()
                        else:
                            pltpu.make_async_copy(
                                rows[b2], agg_sh.at[dst_v.at[g]],
                                sem_s[b2]).wait()
                        ga = jnp.where(g + nbuf - 1 < chg, g + nbuf - 1, 0)
                        pltpu.async_copy(m_hbm.at[src_v.at[ga]], rows[b2],
                                         sem_g[b2])
                    return carry

                lax.fori_loop(0, chg // nbuf, qstep, 0)
                for b in range(nbuf - 1):
                    pltpu.make_async_copy(
                        m_hbm.at[src_v.at[0]], rows[b], sem_g[b]).wait()
                pltpu.make_async_copy(
                    rows[nbuf - 1], agg_sh.at[dst_v.at[0]],
                    sem_s[nbuf - 1]).wait()
                return carry

            lax.fori_loop(0, nch, chunk, 0)

        @pl.when(cid == 0)
        def _():
            main_loop(m0_hbm)

        @pl.when(cid == 1)
        def _():
            main_loop(m1_hbm)

        plsc.subcore_barrier()

        @pl.when(cid == 0)
        def _():
            pltpu.sync_copy(agg_sh.at[pl.ds(base, ROWS_PER_SUB)],
                            out0.at[pl.ds(base, ROWS_PER_SUB)])

        @pl.when(cid == 1)
        def _():
            pltpu.sync_copy(agg_sh.at[pl.ds(base, ROWS_PER_SUB)],
                            out1.at[pl.ds(base, ROWS_PER_SUB)])

    return edge_agg


def _proj0_body(x_ref, wrel_ref, wroot_ref, m_ref, r_ref):
    x = x_ref[...]
    m_ref[...] = jnp.dot(x, wrel_ref[...], preferred_element_type=jnp.float32)
    r_ref[...] = jnp.dot(x, wroot_ref[...], preferred_element_type=jnp.float32)


def _l0post_body(a0_ref, a1_ref, r_ref, b_ref, out_ref):
    out_ref[...] = _sigmoid(
        a0_ref[:N_NODES] + a1_ref[:N_NODES] + r_ref[...] + b_ref[0, :])


def _dense_body(mode_in, split_out, wout, *refs):
    if mode_in == "edge":
        a0_ref, a1_ref, h0_ref, w_ref, r_ref, b_ref = refs[:6]
        out_refs = refs[6:]
        z = jnp.dot(a0_ref[:N_NODES] + a1_ref[:N_NODES], w_ref[...],
                    preferred_element_type=jnp.float32)
        z = z + jnp.dot(h0_ref[...], r_ref[...],
                        preferred_element_type=jnp.float32)
    else:
        a0_ref, a1_ref, h0_ref, h1_ref, w_ref, r_ref, b_ref = refs[:7]
        out_refs = refs[7:]
        w = w_ref[...]
        r = r_ref[...]
        z = (jnp.dot(a0_ref[:N_NODES], w[:COL_W],
                     preferred_element_type=jnp.float32)
             + jnp.dot(a1_ref[:N_NODES], w[COL_W:],
                       preferred_element_type=jnp.float32)
             + jnp.dot(h0_ref[...], r[:COL_W],
                       preferred_element_type=jnp.float32)
             + jnp.dot(h1_ref[...], r[COL_W:],
                       preferred_element_type=jnp.float32))
    h = _sigmoid(z + b_ref[0, :])
    if not split_out:
        out_refs[0][...] = h
    else:
        n = h.shape[0]
        out_refs[0][...] = h[:, :COL_W]
        hi = h[:, COL_W:]
        if wout - COL_W < COL_W:
            hi = jnp.concatenate(
                [hi, jnp.zeros((n, 2 * COL_W - wout), jnp.float32)], axis=1)
        out_refs[1][...] = hi


def _final_body(hlo_ref, hhi_ref, wp_ref, gamma_ref, beta_ref, fc1a_ref,
                fc1b_ref, fc2w_ref, fc2b_ref, out_ref):
    h = jnp.concatenate([hlo_ref[...], hhi_ref[...]], axis=1)
    n = h.shape[0]
    cf = h.shape[1]
    mean = jnp.sum(h, axis=0) / n
    var = jnp.sum(h * h, axis=0) / n - mean * mean
    inv = gamma_ref[0, :] / jnp.sqrt(var + 1e-5)

    wp = wp_ref[...]
    cols = lax.broadcasted_iota(jnp.int32, (NB_WAY, n), 1)
    onehot = jnp.where(cols == wp, 1.0, 0.0).astype(jnp.float32)
    sel = jnp.dot(onehot, h, preferred_element_type=jnp.float32)
    sel = (sel - mean) * inv + beta_ref[0, :]

    fc1 = fc1a_ref[...]
    A = jnp.dot(sel, fc1[:cf], preferred_element_type=jnp.float32)
    A = A + fc1b_ref[0, :]
    B = jnp.dot(sel, fc1[cf:], preferred_element_type=jnp.float32)

    fc2w = fc2w_ref[...]
    fc2b = fc2b_ref[0, :]
    CH = 8
    rows = []
    for c in range(NB_WAY // CH):
        Ac = A[c * CH:(c + 1) * CH]
        hid = Ac[:, None, :] + B[None, :, :]
        hid = _sigmoid(hid).reshape(CH * NB_WAY, A.shape[1])
        p = jnp.dot(hid, fc2w, preferred_element_type=jnp.float32) + fc2b
        rows.append(jnp.max(p, axis=1).reshape(CH, NB_WAY))
    t = _sigmoid(jnp.concatenate(rows, axis=0))
    eye = (lax.broadcasted_iota(jnp.int32, (NB_WAY, NB_WAY), 0)
           == lax.broadcasted_iota(jnp.int32, (NB_WAY, NB_WAY), 1))
    t = jnp.where(eye, 0.0, t)
    out_ref[0, :, :] = t.T


def kernel(x, edge_index, edge_attr, waypoints, conv_params, bn_gamma,
           bn_beta, fc1_W, fc1_b, fc2_W, fc2_b):
    src = edge_index[0].astype(jnp.int32)
    dst = edge_index[1].astype(jnp.int32)
    npad = E_PAD - src.shape[0]
    pad_idx = (jnp.arange(npad, dtype=jnp.int32) * 97) % N_NODES
    srcp = jnp.concatenate([src, pad_idx])
    dstp = jnp.concatenate([dst, pad_idx])
    attrp = jnp.concatenate([edge_attr, jnp.zeros((npad,), jnp.float32)])
    edges_e = tuple(a.reshape(2, NSUB, GRP_E, GB) for a in (srcp, dstp, attrp))
    edges_c = tuple(
        jnp.broadcast_to(a.reshape(1, NSUB, GRP_C, GB), (2, NSUB, GRP_C, GB))
        for a in (srcp, dstp, attrp))

    W0, b0, R0 = conv_params[0]
    s0 = W0.shape[1]
    wp0 = _pad16(s0)
    W0p = jnp.pad(W0, ((0, 0), (0, wp0 - s0)))
    R0p = jnp.pad(R0, ((0, 0), (0, wp0 - s0)))
    b0p = jnp.pad(b0, (0, wp0 - s0)).reshape(1, wp0)
    m0, r0 = pl.pallas_call(
        _proj0_body,
        out_shape=[jax.ShapeDtypeStruct((N_NODES, wp0), jnp.float32),
                   jax.ShapeDtypeStruct((N_NODES, wp0), jnp.float32)],
    )(x, W0p, R0p)
    a0, a1 = _make_edge_agg(wp0, GRP_E)(m0, m0, *edges_e)
    h = pl.pallas_call(
        _l0post_body,
        out_shape=jax.ShapeDtypeStruct((N_NODES, wp0), jnp.float32),
    )(a0, a1, r0, b0p)
    h_parts = (h,)
    wprev = wp0

    nlayers = len(conv_params)
    for i in range(1, nlayers):
        W_rel, b_rel, W_root = conv_params[i]
        prev, size = W_rel.shape
        wout = _pad16(size)
        mode_in = "edge" if wprev <= 128 else "col"
        split_out = (i < nlayers - 1 and wout > 128) or i == nlayers - 1
        win = wprev if mode_in == "edge" else 2 * COL_W
        Wp = jnp.pad(W_rel, ((0, win - prev), (0, wout - size)))
        Rp = jnp.pad(W_root, ((0, win - prev), (0, wout - size)))
        bp = jnp.pad(b_rel, (0, wout - size)).reshape(1, wout)
        if mode_in == "edge":
            a0, a1 = _make_edge_agg(wprev, GRP_E)(
                h_parts[0], h_parts[0], *edges_e)
        else:
            a0, a1 = _make_edge_agg(COL_W, GRP_C)(
                h_parts[0], h_parts[1], *edges_c)
        if split_out:
            out_shape = [
                jax.ShapeDtypeStruct((N_NODES, COL_W), jnp.float32),
                jax.ShapeDtypeStruct((N_NODES, COL_W), jnp.float32)]
        else:
            out_shape = [jax.ShapeDtypeStruct((N_NODES, wout), jnp.float32)]
        h_parts = pl.pallas_call(
            functools.partial(_dense_body, mode_in, split_out, wout),
            out_shape=out_shape,
        )(a0, a1, *h_parts, Wp, Rp, bp)
        h_parts = tuple(h_parts)
        wprev = wout

    wp2 = waypoints.astype(jnp.int32).reshape(NB_WAY, 1)
    cf = 2 * COL_W
    fc2w_p = jnp.pad(fc2_W, ((0, 0), (0, 28)))
    fc2b_p = jnp.pad(fc2_b, (0, 28), constant_values=-jnp.inf)
    return pl.pallas_call(
        _final_body,
        out_shape=jax.ShapeDtypeStruct((1, NB_WAY, NB_WAY), jnp.float32),
    )(h_parts[0], h_parts[1], wp2, bn_gamma.reshape(1, cf),
      bn_beta.reshape(1, cf), fc1_W, fc1_b.reshape(1, -1), fc2w_p,
      fc2b_p.reshape(1, -1))
